# Initial kernel scaffold; baseline (speedup 1.0000x reference)
import jax
import jax.numpy as jnp
from jax.experimental import pallas as pl


def kernel(z_e, W):
    raise NotImplementedError("write your pallas kernel here")


# speedup vs baseline: 1.4536x; 1.4536x over previous
import functools

import jax
import jax.numpy as jnp
from jax import lax
from jax.experimental import pallas as pl
from jax.experimental.pallas import tpu as pltpu
from jax.experimental.pallas import tpu_sc as plsc

N_ROWS = 16384
N_CODES = 1024
DIM = 256
BR = 512
COMMIT = 0.25


def _tc_body(z_ref, w_ref, idx_ref, loss_ref, acc_ref):
    z = z_ref[...]
    w = w_ref[...]
    z2 = jnp.sum(z * z, axis=1, keepdims=True)
    w2 = jnp.sum(w * w, axis=1)
    zw = lax.dot_general(z, w, (((1,), (1,)), ((), ())),
                         preferred_element_type=jnp.float32)
    dist = z2 + w2 - 2.0 * zw
    m = jnp.min(dist, axis=1)
    iota = lax.broadcasted_iota(jnp.int32, dist.shape, 1)
    idx = jnp.min(jnp.where(dist == m[:, None], iota, N_CODES), axis=1)
    idx_ref[...] = idx
    step = pl.program_id(0)

    @pl.when(step == 0)
    def _init():
        acc_ref[0] = 0.0

    acc_ref[0] += jnp.sum(m)

    @pl.when(step == pl.num_programs(0) - 1)
    def _fin():
        mean1 = acc_ref[0] / jnp.float32(N_ROWS * DIM)
        loss_ref[0, 0] = mean1 + jnp.float32(COMMIT) * mean1


def _tc_argmin(z, W):
    grid = N_ROWS // BR
    return pl.pallas_call(
        _tc_body,
        grid=(grid,),
        in_specs=[
            pl.BlockSpec((BR, DIM), lambda i: (i, 0)),
            pl.BlockSpec((N_CODES, DIM), lambda i: (0, 0)),
        ],
        out_specs=[
            pl.BlockSpec((BR,), lambda i: (i,)),
            pl.BlockSpec(memory_space=pltpu.SMEM),
        ],
        out_shape=[
            jax.ShapeDtypeStruct((N_ROWS,), jnp.int32),
            jax.ShapeDtypeStruct((1, 1), jnp.float32),
        ],
        scratch_shapes=[pltpu.SMEM((1,), jnp.float32)],
    )(z, W)


_SC_CORES = 2
_SC_SUBCORES = 16
_NW = _SC_CORES * _SC_SUBCORES
_B_PER_W = N_ROWS // _NW
_CHUNK = 128
_NCH = _B_PER_W // _CHUNK


def _sc_gather_body(w_hbm, idx_hbm, out_hbm, idx_v, buf0, buf1, sem0, sem1):
    wid = lax.axis_index("s") * _SC_CORES + lax.axis_index("c")
    base = wid * _B_PER_W
    pltpu.sync_copy(idx_hbm.at[pl.ds(base, _B_PER_W)], idx_v)
    bufs = (buf0, buf1)
    sems = (sem0, sem1)
    copies = [None] * _NCH
    for c in range(min(2, _NCH)):
        copies[c] = pltpu.async_copy(
            w_hbm.at[idx_v.at[pl.ds(c * _CHUNK, _CHUNK)]], bufs[c % 2],
            sems[c % 2])
    for c in range(_NCH):
        copies[c].wait()
        pltpu.sync_copy(bufs[c % 2],
                        out_hbm.at[pl.ds(base + c * _CHUNK, _CHUNK)])
        nxt = c + 2
        if nxt < _NCH:
            copies[nxt] = pltpu.async_copy(
                w_hbm.at[idx_v.at[pl.ds(nxt * _CHUNK, _CHUNK)]],
                bufs[nxt % 2], sems[nxt % 2])


@functools.cache
def _sc_gather_kernel():
    return pl.kernel(
        _sc_gather_body,
        out_type=jax.ShapeDtypeStruct((N_ROWS, DIM), jnp.float32),
        mesh=plsc.VectorSubcoreMesh(core_axis_name="c", subcore_axis_name="s"),
        scratch_types=[
            pltpu.VMEM((_B_PER_W,), jnp.int32),
            pltpu.VMEM((_CHUNK, DIM), jnp.float32),
            pltpu.VMEM((_CHUNK, DIM), jnp.float32),
            pltpu.SemaphoreType.DMA,
            pltpu.SemaphoreType.DMA,
        ],
    )


def kernel(z_e, W):
    indices, loss = _tc_argmin(z_e, W)
    z_q_st = _sc_gather_kernel()(W, indices)
    return (z_q_st, indices, loss.reshape(()))


# speedup vs baseline: 1.6248x; 1.1178x over previous
import functools

import jax
import jax.numpy as jnp
from jax import lax
from jax.experimental import pallas as pl
from jax.experimental.pallas import tpu as pltpu
from jax.experimental.pallas import tpu_sc as plsc

N_ROWS = 16384
N_CODES = 1024
DIM = 256
BR = 512
COMMIT = 0.25


def _tc_body(z_ref, w_ref, idx_ref, loss_ref, w2_ref, w2x_ref, acc_ref):
    step = pl.program_id(0)
    z = z_ref[...]

    @pl.when(step == 0)
    def _init():
        w = w_ref[...]
        acc_ref[0] = 0.0
        w2_ref[...] = jnp.sum(w * w, axis=1)[None, :]
        w2x_ref[...] = w + w

    z2 = jnp.sum(z * z, axis=1, keepdims=True)
    zw2 = lax.dot_general(z, w2x_ref[...], (((1,), (1,)), ((), ())),
                          preferred_element_type=jnp.float32)
    dist = (z2 + w2_ref[...]) - zw2
    m = jnp.min(dist, axis=1, keepdims=True)
    iota = lax.broadcasted_iota(jnp.int32, (1, N_CODES), 1).astype(jnp.float32)
    idx_f = jnp.min(jnp.where(dist == m, iota, float(N_CODES)),
                    axis=1, keepdims=True)
    idx_ref[...] = idx_f.astype(jnp.int32)

    acc_ref[0] += jnp.sum(m)

    @pl.when(step == pl.num_programs(0) - 1)
    def _fin():
        mean1 = acc_ref[0] / jnp.float32(N_ROWS * DIM)
        loss_ref[0, 0] = mean1 + jnp.float32(COMMIT) * mean1


def _tc_argmin(z, W):
    grid = N_ROWS // BR
    return pl.pallas_call(
        _tc_body,
        grid=(grid,),
        in_specs=[
            pl.BlockSpec((BR, DIM), lambda i: (i, 0)),
            pl.BlockSpec((N_CODES, DIM), lambda i: (0, 0)),
        ],
        out_specs=[
            pl.BlockSpec((BR, 1), lambda i: (i, 0)),
            pl.BlockSpec(memory_space=pltpu.SMEM),
        ],
        out_shape=[
            jax.ShapeDtypeStruct((N_ROWS, 1), jnp.int32),
            jax.ShapeDtypeStruct((1, 1), jnp.float32),
        ],
        scratch_shapes=[pltpu.VMEM((1, N_CODES), jnp.float32),
                        pltpu.VMEM((N_CODES, DIM), jnp.float32),
                        pltpu.SMEM((1,), jnp.float32)],
    )(z, W)


_SC_CORES = 2
_SC_SUBCORES = 16
_NW = _SC_CORES * _SC_SUBCORES
_B_PER_W = N_ROWS // _NW
_CHUNK = 128
_NCH = _B_PER_W // _CHUNK


def _sc_gather_body(w_hbm, idx_hbm, out_hbm, idx_v, buf0, buf1, sem0, sem1):
    wid = lax.axis_index("s") * _SC_CORES + lax.axis_index("c")
    base = wid * _B_PER_W
    pltpu.sync_copy(idx_hbm.at[pl.ds(base, _B_PER_W)], idx_v)
    bufs = (buf0, buf1)
    sems = (sem0, sem1)
    copies = [None] * _NCH
    for c in range(min(2, _NCH)):
        copies[c] = pltpu.async_copy(
            w_hbm.at[idx_v.at[pl.ds(c * _CHUNK, _CHUNK)]], bufs[c % 2],
            sems[c % 2])
    for c in range(_NCH):
        copies[c].wait()
        pltpu.sync_copy(bufs[c % 2],
                        out_hbm.at[pl.ds(base + c * _CHUNK, _CHUNK)])
        nxt = c + 2
        if nxt < _NCH:
            copies[nxt] = pltpu.async_copy(
                w_hbm.at[idx_v.at[pl.ds(nxt * _CHUNK, _CHUNK)]],
                bufs[nxt % 2], sems[nxt % 2])


@functools.cache
def _sc_gather_kernel():
    return pl.kernel(
        _sc_gather_body,
        out_type=jax.ShapeDtypeStruct((N_ROWS, DIM), jnp.float32),
        mesh=plsc.VectorSubcoreMesh(core_axis_name="c", subcore_axis_name="s"),
        scratch_types=[
            pltpu.VMEM((_B_PER_W,), jnp.int32),
            pltpu.VMEM((_CHUNK, DIM), jnp.float32),
            pltpu.VMEM((_CHUNK, DIM), jnp.float32),
            pltpu.SemaphoreType.DMA,
            pltpu.SemaphoreType.DMA,
        ],
    )


def kernel(z_e, W):
    indices2d, loss = _tc_argmin(z_e, W)
    indices = indices2d.reshape(N_ROWS)
    z_q_st = _sc_gather_kernel()(W, indices)
    return (z_q_st, indices, loss.reshape(()))


# speedup vs baseline: 1.9764x; 1.2164x over previous
import functools

import jax
import jax.numpy as jnp
from jax import lax
from jax.experimental import pallas as pl
from jax.experimental.pallas import tpu as pltpu
from jax.experimental.pallas import tpu_sc as plsc

N_ROWS = 16384
N_CODES = 1024
DIM = 256
BR = 4096
COMMIT = 0.25


def _tc_body(z_ref, w_ref, idx_ref, loss_ref, w2_ref, w2x_ref, acc_ref):
    step = pl.program_id(0)
    z = z_ref[...]

    @pl.when(step == 0)
    def _init():
        w = w_ref[...]
        acc_ref[0] = 0.0
        w2_ref[...] = jnp.sum(w * w, axis=1)[None, :]
        w2x_ref[...] = w + w

    z2 = jnp.sum(z * z, axis=1, keepdims=True)
    zw2 = lax.dot_general(z, w2x_ref[...], (((1,), (1,)), ((), ())),
                          preferred_element_type=jnp.float32)
    dist = (z2 + w2_ref[...]) - zw2
    m = jnp.min(dist, axis=1, keepdims=True)
    iota = lax.broadcasted_iota(jnp.int32, (1, N_CODES), 1).astype(jnp.float32)
    idx_f = jnp.min(jnp.where(dist == m, iota, float(N_CODES)),
                    axis=1, keepdims=True)
    idx_ref[...] = idx_f.astype(jnp.int32)

    acc_ref[0] += jnp.sum(m)

    @pl.when(step == pl.num_programs(0) - 1)
    def _fin():
        mean1 = acc_ref[0] / jnp.float32(N_ROWS * DIM)
        loss_ref[0, 0] = mean1 + jnp.float32(COMMIT) * mean1


def _tc_argmin(z, W):
    grid = N_ROWS // BR
    return pl.pallas_call(
        _tc_body,
        grid=(grid,),
        in_specs=[
            pl.BlockSpec((BR, DIM), lambda i: (i, 0)),
            pl.BlockSpec((N_CODES, DIM), lambda i: (0, 0)),
        ],
        out_specs=[
            pl.BlockSpec((BR, 1), lambda i: (i, 0)),
            pl.BlockSpec(memory_space=pltpu.SMEM),
        ],
        out_shape=[
            jax.ShapeDtypeStruct((N_ROWS, 1), jnp.int32),
            jax.ShapeDtypeStruct((1, 1), jnp.float32),
        ],
        scratch_shapes=[pltpu.VMEM((1, N_CODES), jnp.float32),
                        pltpu.VMEM((N_CODES, DIM), jnp.float32),
                        pltpu.SMEM((1,), jnp.float32)],
    )(z, W)


_SC_CORES = 2
_SC_SUBCORES = 16
_NW = _SC_CORES * _SC_SUBCORES
_B_PER_W = N_ROWS // _NW
_CHUNK = 128
_NCH = _B_PER_W // _CHUNK


def _sc_gather_body(w_hbm, idx_hbm, out_hbm, idx_v, buf0, buf1, sem0, sem1):
    wid = lax.axis_index("s") * _SC_CORES + lax.axis_index("c")
    base = wid * _B_PER_W
    pltpu.sync_copy(idx_hbm.at[pl.ds(base, _B_PER_W)], idx_v)
    bufs = (buf0, buf1)
    sems = (sem0, sem1)
    copies = [None] * _NCH
    for c in range(min(2, _NCH)):
        copies[c] = pltpu.async_copy(
            w_hbm.at[idx_v.at[pl.ds(c * _CHUNK, _CHUNK)]], bufs[c % 2],
            sems[c % 2])
    for c in range(_NCH):
        copies[c].wait()
        pltpu.sync_copy(bufs[c % 2],
                        out_hbm.at[pl.ds(base + c * _CHUNK, _CHUNK)])
        nxt = c + 2
        if nxt < _NCH:
            copies[nxt] = pltpu.async_copy(
                w_hbm.at[idx_v.at[pl.ds(nxt * _CHUNK, _CHUNK)]],
                bufs[nxt % 2], sems[nxt % 2])


@functools.cache
def _sc_gather_kernel():
    return pl.kernel(
        _sc_gather_body,
        out_type=jax.ShapeDtypeStruct((N_ROWS, DIM), jnp.float32),
        mesh=plsc.VectorSubcoreMesh(core_axis_name="c", subcore_axis_name="s"),
        scratch_types=[
            pltpu.VMEM((_B_PER_W,), jnp.int32),
            pltpu.VMEM((_CHUNK, DIM), jnp.float32),
            pltpu.VMEM((_CHUNK, DIM), jnp.float32),
            pltpu.SemaphoreType.DMA,
            pltpu.SemaphoreType.DMA,
        ],
    )


def kernel(z_e, W):
    indices2d, loss = _tc_argmin(z_e, W)
    indices = indices2d.reshape(N_ROWS)
    z_q_st = _sc_gather_kernel()(W, indices)
    return (z_q_st, indices, loss.reshape(()))
